# Initial kernel scaffold; baseline (speedup 1.0000x reference)
import jax
import jax.numpy as jnp
from jax.experimental import pallas as pl


def kernel(node_features, edge_index, W1, b1, W2, b2):
    raise NotImplementedError("write your pallas kernel here")


# speedup vs baseline: 12.0174x; 12.0174x over previous
import functools

import jax
import jax.numpy as jnp
from jax import lax
from jax.experimental import pallas as pl
from jax.experimental.pallas import tpu as pltpu
from jax.experimental.pallas import tpu_sc as plsc

NNODE = 10000
NEDGE = 320000
DIM = 128

NC = 2
NS = 16
NW = NC * NS
LANES = 16

NPAD = 10240
KCH = 128
CPT = NEDGE // (NW * KCH)
if NEDGE % (NW * KCH):
    CPT += 1
EPT = CPT * KCH
EPAD = EPT * NW
RPT = NPAD // NS

_mesh = plsc.VectorSubcoreMesh(
    core_axis_name="c", subcore_axis_name="s", num_cores=NC, num_subcores=NS)


@functools.partial(
    pl.kernel,
    out_type=jax.ShapeDtypeStruct((NW, NPAD), jnp.float32),
    mesh=_mesh,
    scratch_types=[
        pltpu.VMEM((EPT,), jnp.int32),
        pltpu.VMEM((NPAD,), jnp.float32),
    ],
    compiler_params=pltpu.CompilerParams(needs_layout_passes=False),
)
def _deg_kernel(dst_hbm, out_hbm, dst_v, deg_v):
    cid = lax.axis_index("c")
    sid = lax.axis_index("s")
    wid = sid * NC + cid
    pltpu.sync_copy(dst_hbm.at[pl.ds(wid * EPT, EPT)], dst_v)

    @pl.loop(0, NPAD // LANES)
    def _zero(i):
        deg_v[pl.ds(i * LANES, LANES)] = jnp.zeros((LANES,), jnp.float32)

    ones = jnp.ones((LANES,), jnp.float32)

    @pl.loop(0, EPT // LANES)
    def _acc(i):
        idx = dst_v[pl.ds(i * LANES, LANES)]
        plsc.addupdate_scatter(deg_v, [idx], ones)

    pltpu.sync_copy(deg_v, out_hbm.at[wid])


@functools.partial(
    pl.kernel,
    out_type=jax.ShapeDtypeStruct((NC, NPAD, DIM), jnp.float32),
    mesh=_mesh,
    scratch_types=[
        pltpu.VMEM((CPT, KCH), jnp.int32),
        pltpu.VMEM((CPT, KCH), jnp.int32),
        pltpu.VMEM((KCH, DIM), jnp.float32),
        pltpu.VMEM_SHARED((NPAD, DIM), jnp.float32),
        pltpu.SemaphoreType.DMA,
    ],
    compiler_params=pltpu.CompilerParams(needs_layout_passes=False),
)
def _spmm_kernel(g_hbm, src_hbm, dst_hbm, zero_hbm, out_hbm,
                 src_v, dst_v, rows_v, acc_sh, sem):
    cid = lax.axis_index("c")
    sid = lax.axis_index("s")
    wid = sid * NC + cid
    pltpu.sync_copy(src_hbm.at[wid], src_v)
    pltpu.sync_copy(dst_hbm.at[wid], dst_v)

    row0 = sid * RPT

    @pl.when(cid == 0)
    def _():
        pltpu.sync_copy(g_hbm.at[pl.ds(row0, RPT)], acc_sh.at[pl.ds(row0, RPT)])

    @pl.when(cid != 0)
    def _():
        pltpu.sync_copy(zero_hbm.at[pl.ds(row0, RPT)],
                        acc_sh.at[pl.ds(row0, RPT)])

    plsc.subcore_barrier()

    @pl.loop(0, CPT)
    def _chunk(j):
        pltpu.async_copy(g_hbm.at[src_v.at[j]], rows_v, sem).wait()
        pltpu.sync_copy(rows_v, acc_sh.at[dst_v.at[j]], add=True)

    plsc.subcore_barrier()
    pltpu.sync_copy(acc_sh.at[pl.ds(row0, RPT)],
                    out_hbm.at[cid, pl.ds(row0, RPT)])


_RB = 512


def _mm1_body(x_ref, w_ref, degp_ref, g_ref, dinv_ref):
    deg = jnp.sum(degp_ref[...], axis=0) + 1.0
    dinv = lax.rsqrt(deg)
    h = jnp.dot(x_ref[...], w_ref[...], preferred_element_type=jnp.float32)
    g_ref[...] = h * dinv[:, None]
    dinv_ref[...] = dinv[None, :]


def _mid_body(t_ref, dinv_ref, w_ref, b_ref, g_ref):
    t = t_ref[0] + t_ref[1]
    dinv = dinv_ref[0]
    y = t * dinv[:, None] + b_ref[...]
    h = jnp.dot(y, w_ref[...], preferred_element_type=jnp.float32)
    g_ref[...] = h * dinv[:, None]


def _fin_body(t_ref, dinv_ref, b_ref, o_ref):
    t = t_ref[0] + t_ref[1]
    o_ref[...] = t * dinv_ref[0][:, None] + b_ref[...]


_GRID = NPAD // _RB

_mm1 = pl.pallas_call(
    _mm1_body,
    grid=(_GRID,),
    in_specs=[
        pl.BlockSpec((_RB, DIM), lambda i: (i, 0)),
        pl.BlockSpec((DIM, DIM), lambda i: (0, 0)),
        pl.BlockSpec((NW, _RB), lambda i: (0, i)),
    ],
    out_specs=[
        pl.BlockSpec((_RB, DIM), lambda i: (i, 0)),
        pl.BlockSpec((1, _RB), lambda i: (0, i)),
    ],
    out_shape=[
        jax.ShapeDtypeStruct((NPAD, DIM), jnp.float32),
        jax.ShapeDtypeStruct((1, NPAD), jnp.float32),
    ],
)

_mid = pl.pallas_call(
    _mid_body,
    grid=(_GRID,),
    in_specs=[
        pl.BlockSpec((NC, _RB, DIM), lambda i: (0, i, 0)),
        pl.BlockSpec((1, _RB), lambda i: (0, i)),
        pl.BlockSpec((DIM, DIM), lambda i: (0, 0)),
        pl.BlockSpec((1, DIM), lambda i: (0, 0)),
    ],
    out_specs=pl.BlockSpec((_RB, DIM), lambda i: (i, 0)),
    out_shape=jax.ShapeDtypeStruct((NPAD, DIM), jnp.float32),
)

_fin = pl.pallas_call(
    _fin_body,
    grid=(_GRID,),
    in_specs=[
        pl.BlockSpec((NC, _RB, DIM), lambda i: (0, i, 0)),
        pl.BlockSpec((1, _RB), lambda i: (0, i)),
        pl.BlockSpec((1, DIM), lambda i: (0, 0)),
    ],
    out_specs=pl.BlockSpec((_RB, DIM), lambda i: (i, 0)),
    out_shape=jax.ShapeDtypeStruct((NPAD, DIM), jnp.float32),
)


def kernel(node_features, edge_index, W1, b1, W2, b2):
    xpad = jnp.zeros((NPAD, DIM), jnp.float32).at[:NNODE].set(node_features)
    pad_e = jnp.full((EPAD - NEDGE,), NPAD - 1, jnp.int32)
    src = jnp.concatenate([edge_index[0], pad_e]).reshape(NW, CPT, KCH)
    dst = jnp.concatenate([edge_index[1], pad_e]).reshape(NW, CPT, KCH)
    zeros_nd = jnp.zeros((NPAD, DIM), jnp.float32)

    degp = _deg_kernel(dst.reshape(EPAD))
    g1, dinv = _mm1(xpad, W1, degp)
    t1 = _spmm_kernel(g1, src, dst, zeros_nd)
    g2 = _mid(t1, dinv, W2, b1.reshape(1, DIM))
    t2 = _spmm_kernel(g2, src, dst, zeros_nd)
    out = _fin(t2, dinv, b2.reshape(1, DIM))
    return out[:NNODE]
